# baseline (speedup 1.0000x reference)
import functools

import jax
import jax.numpy as jnp
from jax import lax
from jax.experimental import pallas as pl
from jax.experimental.pallas import tpu as pltpu
from jax.experimental.pallas import tpu_sc as plsc

NUM_CORES = 2
NUM_SUBCORES = 16
NUM_WORKERS = NUM_CORES * NUM_SUBCORES
LANES = 16
CHUNK = 128


@functools.cache
def _build(batch, n_rows, dim):
    b_per_w = batch // NUM_WORKERS
    n_chunks = b_per_w // CHUNK
    mesh = plsc.VectorSubcoreMesh(
        core_axis_name="c", subcore_axis_name="s",
        num_cores=NUM_CORES, num_subcores=NUM_SUBCORES)

    @functools.partial(
        pl.kernel,
        out_type=jax.ShapeDtypeStruct((batch, dim), jnp.float32),
        mesh=mesh,
        scratch_types=[
            pltpu.VMEM((n_chunks, CHUNK), jnp.int32),
            pltpu.VMEM((n_chunks, CHUNK), jnp.int32),
            pltpu.VMEM((b_per_w, dim), jnp.float32),
            pltpu.VMEM((b_per_w, dim), jnp.float32),
            pltpu.SemaphoreType.DMA,
        ],
        compiler_params=pltpu.CompilerParams(use_tc_tiling_on_sc=False),
    )
    def gmf(idx_a_hbm, idx_b_hbm, table_hbm, out_hbm,
            idx_a_v, idx_b_v, rows_a_v, rows_b_v, sem):
        wid = lax.axis_index("s") * NUM_CORES + lax.axis_index("c")
        base = wid * b_per_w
        pltpu.sync_copy(idx_a_hbm.at[wid], idx_a_v)
        pltpu.sync_copy(idx_b_hbm.at[wid], idx_b_v)
        copies = []
        for j in range(n_chunks):
            dst = pl.ds(j * CHUNK, CHUNK)
            copies.append(pltpu.async_copy(
                table_hbm.at[idx_a_v.at[j]], rows_a_v.at[dst], sem))
            copies.append(pltpu.async_copy(
                table_hbm.at[idx_b_v.at[j]], rows_b_v.at[dst], sem))
        for c in copies:
            c.wait()

        def mul_row(i, _):
            rows_a_v[i, :] = rows_a_v[i, :] * rows_b_v[i, :]
            return 0

        lax.fori_loop(0, b_per_w, mul_row, 0, unroll=8)
        pltpu.sync_copy(rows_a_v, out_hbm.at[pl.ds(base, b_per_w)])

    return gmf


def kernel(input_plylst, input_item, table_plylst, table_item):
    batch = input_plylst.shape[0]
    n_rows, dim = table_plylst.shape
    b_per_w = batch // NUM_WORKERS
    idx_a = input_plylst.astype(jnp.int32).reshape(
        NUM_WORKERS, b_per_w // CHUNK, CHUNK)
    idx_b = input_item.astype(jnp.int32).reshape(
        NUM_WORKERS, b_per_w // CHUNK, CHUNK)
    return _build(batch, n_rows, dim)(idx_a, idx_b, table_plylst)
